# Initial kernel scaffold; baseline (speedup 1.0000x reference)
import jax
import jax.numpy as jnp
from jax.experimental import pallas as pl


def kernel(x, edge_index, edge_attr, pool_edge_index, pool_edge_attr, n_fine, W_root, W_agg, b, we):
    raise NotImplementedError("write your pallas kernel here")


# speedup vs baseline: 2.8584x; 2.8584x over previous
import functools

import jax
import jax.numpy as jnp
from jax import lax
from jax.experimental import pallas as pl
from jax.experimental.pallas import tpu as pltpu
from jax.experimental.pallas import tpu_sc as plsc

N = 10000
D = 128
E = 320000
EP = 80000
NF = 40000
L = 16
NC = 2
NS = 16
GA = E // 128
GC = EP // 128
NW = NC * NS


def _mesh():
    return plsc.VectorSubcoreMesh(core_axis_name="c", subcore_axis_name="s")


@functools.partial(
    pl.kernel,
    out_type=jax.ShapeDtypeStruct((NC, N, D), jnp.float32),
    mesh=_mesh(),
    scratch_types=[
        pltpu.VMEM((128,), jnp.int32),
        pltpu.VMEM((128,), jnp.int32),
        pltpu.VMEM((128,), jnp.float32),
        pltpu.VMEM((128,), jnp.float32),
        pltpu.VMEM((128,), jnp.float32),
        pltpu.VMEM((128,), jnp.float32),
        pltpu.VMEM((128, D), jnp.float32),
        pltpu.VMEM((80, D), jnp.float32),
        pltpu.VMEM((16,), jnp.float32),
        pltpu.VMEM_SHARED((N, D), jnp.float32),
    ],
)
def _sc_aggr(x_hbm, src_hbm, dst_hbm, ea_hbm, we_hbm, out_hbm,
             src_v, dst_v, ea0_v, ea1_v, ea2_v, ea3_v, rows_v, zero_v,
             we_v, acc):
    c = lax.axis_index("c")
    s = lax.axis_index("s")
    wid = s * NC + c

    zf = jnp.zeros((L,), jnp.float32)

    def zrow(r, carry):
        for k in range(D // L):
            zero_v[r, pl.ds(k * L, L)] = zf
        return carry

    lax.fori_loop(0, 80, zrow, 0)

    def zchunk(i, carry):
        ch = s + i * NS
        pltpu.sync_copy(zero_v, acc.at[pl.ds(ch * 80, 80), :])
        return carry

    nz = (N // 80 - s + NS - 1) // NS
    lax.fori_loop(0, nz, zchunk, 0)
    pltpu.sync_copy(we_hbm, we_v)
    plsc.subcore_barrier()

    wev = we_v[...]

    ngroups = (GA - wid + NW - 1) // NW

    def group(i, carry):
        g = wid + i * NW
        pltpu.sync_copy(src_hbm.at[g], src_v)
        pltpu.sync_copy(dst_hbm.at[g], dst_v)
        pltpu.sync_copy(ea_hbm.at[0, g], ea0_v)
        pltpu.sync_copy(ea_hbm.at[1, g], ea1_v)
        pltpu.sync_copy(ea_hbm.at[2, g], ea2_v)
        pltpu.sync_copy(ea_hbm.at[3, g], ea3_v)
        pltpu.sync_copy(x_hbm.at[src_v], rows_v)

        def subgroup(q, ecarry):
            wq = (ea0_v[pl.ds(q * L, L)] * wev[0]
                  + ea1_v[pl.ds(q * L, L)] * wev[1]
                  + ea2_v[pl.ds(q * L, L)] * wev[2]
                  + ea3_v[pl.ds(q * L, L)] * wev[3])
            for j2 in range(L):
                bw = lax.broadcast(wq[j2], (L,))
                j = q * L + j2
                for k in range(D // L):
                    rows_v[j, pl.ds(k * L, L)] = (
                        rows_v[j, pl.ds(k * L, L)] * bw)
            return ecarry

        lax.fori_loop(0, 128 // L, subgroup, 0)
        pltpu.sync_copy(rows_v, acc.at[dst_v], add=True)
        return carry

    lax.fori_loop(0, ngroups, group, 0)
    plsc.subcore_barrier()

    def fchunk(i, carry):
        ch = s + i * NS
        pltpu.sync_copy(acc.at[pl.ds(ch * 80, 80), :],
                        out_hbm.at[c, pl.ds(ch * 80, 80), :])
        return carry

    lax.fori_loop(0, nz, fchunk, 0)


RB = 1000


def _tc_body(x_ref, p0_ref, p1_ref, wr_ref, wa_ref, b_ref, o_ref):
    ag = p0_ref[...] + p1_ref[...]
    acc = jnp.dot(x_ref[...], wr_ref[...], preferred_element_type=jnp.float32)
    acc = acc + jnp.dot(ag, wa_ref[...], preferred_element_type=jnp.float32)
    acc = acc + b_ref[...]
    o_ref[...] = jnp.maximum(acc, 0.0)


_tc_dense = pl.pallas_call(
    _tc_body,
    grid=(N // RB,),
    in_specs=[
        pl.BlockSpec((RB, D), lambda i: (i, 0)),
        pl.BlockSpec((RB, D), lambda i: (i, 0)),
        pl.BlockSpec((RB, D), lambda i: (i, 0)),
        pl.BlockSpec((D, D), lambda i: (0, 0)),
        pl.BlockSpec((D, D), lambda i: (0, 0)),
        pl.BlockSpec((1, D), lambda i: (0, 0)),
    ],
    out_specs=pl.BlockSpec((RB, D), lambda i: (i, 0)),
    out_shape=jax.ShapeDtypeStruct((N, D), jnp.float32),
)


@functools.partial(
    pl.kernel,
    out_type=jax.ShapeDtypeStruct((4, NF, 32), jnp.float32),
    mesh=_mesh(),
    compiler_params=pltpu.CompilerParams(use_tc_tiling_on_sc=False),
    scratch_types=[
        pltpu.VMEM((128,), jnp.int32),
        pltpu.VMEM((128,), jnp.int32),
        pltpu.VMEM((128,), jnp.float32),
        pltpu.VMEM((128, 32), jnp.float32),
        pltpu.VMEM((200, 32), jnp.float32),
        pltpu.VMEM_SHARED((NF, 32), jnp.float32),
    ],
)
def _sc_unpool(h_hbm, ps_hbm, fid_hbm, wp_hbm, out_hbm,
               gi_v, fi_v, w_v, rows_v, zero_v, acc):
    c = lax.axis_index("c")
    s = lax.axis_index("s")

    zf = jnp.zeros((L,), jnp.float32)

    def zrow(r, carry):
        zero_v[r, pl.ds(0, L)] = zf
        zero_v[r, pl.ds(L, L)] = zf
        return carry

    lax.fori_loop(0, 200, zrow, 0)
    ngroups = (GC - s + NS - 1) // NS
    nz = (NF // 200 - s + NS - 1) // NS

    def zchunk(i, carry):
        ch = s + i * NS
        pltpu.sync_copy(zero_v, acc.at[pl.ds(ch * 200, 200), :])
        return carry

    for blk in range(2):
        cb = c * 2 + blk
        lax.fori_loop(0, nz, zchunk, 0)
        plsc.subcore_barrier()

        offv = lax.broadcast(cb * N, (L,))

        def group(i, carry):
            g = s + i * NS
            pltpu.sync_copy(ps_hbm.at[g], gi_v)
            pltpu.sync_copy(fid_hbm.at[g], fi_v)
            pltpu.sync_copy(wp_hbm.at[g], w_v)
            for k in range(128 // L):
                gi_v[pl.ds(k * L, L)] = gi_v[pl.ds(k * L, L)] + offv
            pltpu.sync_copy(h_hbm.at[gi_v], rows_v)

            def subgroup(q, ecarry):
                wq = w_v[pl.ds(q * L, L)]
                for j2 in range(L):
                    bw = lax.broadcast(wq[j2], (L,))
                    j = q * L + j2
                    rows_v[j, pl.ds(0, L)] = rows_v[j, pl.ds(0, L)] * bw
                    rows_v[j, pl.ds(L, L)] = rows_v[j, pl.ds(L, L)] * bw
                return ecarry

            lax.fori_loop(0, 128 // L, subgroup, 0)
            pltpu.sync_copy(rows_v, acc.at[fi_v], add=True)
            return carry

        lax.fori_loop(0, ngroups, group, 0)
        plsc.subcore_barrier()

        def fchunk(i, carry):
            ch = s + i * NS
            pltpu.sync_copy(acc.at[pl.ds(ch * 200, 200), :],
                            out_hbm.at[cb, pl.ds(ch * 200, 200), :])
            return carry

        lax.fori_loop(0, nz, fchunk, 0)
        plsc.subcore_barrier()


def kernel(x, edge_index, edge_attr, pool_edge_index, pool_edge_attr,
           n_fine, W_root, W_agg, b, we):
    src = edge_index[0].reshape(GA, 128)
    dst = edge_index[1].reshape(GA, 128)
    ea = edge_attr.T.reshape(4, GA, 128)
    we16 = jnp.zeros((16,), jnp.float32).at[:4].set(we[:, 0])

    p01 = _sc_aggr(x, src, dst, ea, we16)
    h = _tc_dense(x, p01[0], p01[1], W_root, W_agg, b.reshape(1, D))

    hcb = h.reshape(N, 4, 32).transpose(1, 0, 2).reshape(4 * N, 32)
    ps = pool_edge_index[1].reshape(GC, 128)
    fid = jnp.minimum(pool_edge_index[0], n_fine - 1).astype(jnp.int32)
    fid = fid.reshape(GC, 128)
    wp = pool_edge_attr.reshape(GC, 128)
    out_cb = _sc_unpool(hcb, ps, fid, wp)
    return out_cb.transpose(1, 0, 2).reshape(NF, D)


# speedup vs baseline: 2.9854x; 1.0444x over previous
import functools

import jax
import jax.numpy as jnp
from jax import lax
from jax.experimental import pallas as pl
from jax.experimental.pallas import tpu as pltpu
from jax.experimental.pallas import tpu_sc as plsc

N = 10000
D = 128
DH = 64
E = 320000
EP = 80000
NF = 40000
L = 16
NC = 2
NS = 16
NW = NC * NS

CH = 128
EPAD = 327680
CPT_A = EPAD // (NS * CH)
HB_A = CPT_A // 2
EPPAD = 81920
CPT_C = EPPAD // (NS * CH)


def _mesh():
    return plsc.VectorSubcoreMesh(core_axis_name="c", subcore_axis_name="s")


@functools.partial(
    pl.kernel,
    out_type=jax.ShapeDtypeStruct((NC, N, DH), jnp.float32),
    mesh=_mesh(),
    compiler_params=pltpu.CompilerParams(use_tc_tiling_on_sc=False),
    scratch_types=[
        pltpu.VMEM((HB_A, CH), jnp.int32),
        pltpu.VMEM((HB_A, CH), jnp.int32),
        pltpu.VMEM((CPT_A * CH,), jnp.float32),
        pltpu.VMEM((HB_A * CH,), jnp.float32),
        pltpu.VMEM((CH, DH), jnp.float32),
        pltpu.VMEM((CH, DH), jnp.float32),
        pltpu.VMEM((40, DH), jnp.float32),
        pltpu.VMEM((16,), jnp.float32),
        pltpu.VMEM_SHARED((N, DH), jnp.float32),
        pltpu.SemaphoreType.DMA,
        pltpu.SemaphoreType.DMA,
        pltpu.SemaphoreType.DMA,
        pltpu.SemaphoreType.DMA,
    ],
)
def _sc_aggr(x_hbm, src_hbm, dst_hbm, ea_hbm, we_hbm, out_hbm,
             src_v, dst_v, w_v, tmp_v, rows0_v, rows1_v, zero_v, we_v, acc,
             gsem0, gsem1, ssem0, ssem1):
    c = lax.axis_index("c")
    s = lax.axis_index("s")
    rows = (rows0_v, rows1_v)
    gsem = (gsem0, gsem1)
    ssem = (ssem0, ssem1)

    zf = jnp.zeros((L,), jnp.float32)

    def zrow(r, carry):
        for k in range(DH // L):
            zero_v[r, pl.ds(k * L, L)] = zf
        return carry

    lax.fori_loop(0, 40, zrow, 0)

    def zchunk(i, carry):
        ch = s + i * NS
        pltpu.sync_copy(zero_v, acc.at[pl.ds(ch * 40, 40), :])
        return carry

    nz = (N // 40 - s + NS - 1) // NS
    lax.fori_loop(0, nz, zchunk, 0)

    pltpu.sync_copy(we_hbm, we_v)
    wev = we_v[...]
    for k in range(4):
        for hb in range(2):
            pltpu.sync_copy(ea_hbm.at[k, s, pl.ds(hb * HB_A * CH, HB_A * CH)],
                            tmp_v)

            def wacc(i, carry):
                o = hb * (HB_A * CH) + i * L
                t = tmp_v[pl.ds(i * L, L)] * wev[k]
                if k == 0:
                    w_v[pl.ds(o, L)] = t
                else:
                    w_v[pl.ds(o, L)] = w_v[pl.ds(o, L)] + t
                return carry

            lax.fori_loop(0, HB_A * CH // L, wacc, 0)
    plsc.subcore_barrier()

    coff = lax.broadcast(c * N, (L,))

    for hb in range(2):
        pltpu.sync_copy(src_hbm.at[s, pl.ds(hb * HB_A, HB_A), :], src_v)
        pltpu.sync_copy(dst_hbm.at[s, pl.ds(hb * HB_A, HB_A), :], dst_v)

        def shift(r, carry):
            for k in range(CH // L):
                src_v[r, pl.ds(k * L, L)] = src_v[r, pl.ds(k * L, L)] + coff
            return carry

        lax.fori_loop(0, HB_A, shift, 0)

        def gather(ci, b):
            return pltpu.async_copy(x_hbm.at[src_v.at[ci]], rows[b], gsem[b])

        def wait_gather(ci, b):
            pltpu.make_async_copy(x_hbm.at[src_v.at[ci]], rows[b],
                                  gsem[b]).wait()

        def scatter(ci, b):
            return pltpu.async_copy(rows[b], acc.at[dst_v.at[ci]], ssem[b],
                                    add=True)

        def wait_scatter(ci, b):
            pltpu.make_async_copy(rows[b], acc.at[dst_v.at[ci]],
                                  ssem[b]).wait()

        gather(0, 0)
        wbase = hb * (HB_A * CH)

        def pair(t, carry):
            for u in range(2):
                ci = 2 * t + u
                wait_gather(ci, u)

                def subgroup(q, ecarry):
                    wq = w_v[pl.ds(wbase + ci * CH + q * L, L)]
                    for j2 in range(L):
                        bw = lax.broadcast(wq[j2], (L,))
                        j = q * L + j2
                        for k in range(DH // L):
                            rows[u][j, pl.ds(k * L, L)] = (
                                rows[u][j, pl.ds(k * L, L)] * bw)
                    return ecarry

                lax.fori_loop(0, CH // L, subgroup, 0)
                scatter(ci, u)

                if u == 0:
                    @pl.when(t > 0)
                    def _drain0():
                        wait_scatter(ci - 1, 1)
                    gather(ci + 1, 1)
                else:
                    wait_scatter(ci - 1, 0)

                    @pl.when(t < HB_A // 2 - 1)
                    def _pref1():
                        gather(ci + 1, 0)
            return carry

        lax.fori_loop(0, HB_A // 2, pair, 0)
        wait_scatter(HB_A - 1, 1)

    plsc.subcore_barrier()

    def fchunk(i, carry):
        ch = s + i * NS
        pltpu.sync_copy(acc.at[pl.ds(ch * 40, 40), :],
                        out_hbm.at[c, pl.ds(ch * 40, 40), :])
        return carry

    lax.fori_loop(0, nz, fchunk, 0)


RB = 1000


def _tc_body(x_ref, a0_ref, a1_ref, wr_ref, wa0_ref, wa1_ref, b_ref, o_ref):
    acc = jnp.dot(x_ref[...], wr_ref[...], preferred_element_type=jnp.float32)
    acc = acc + jnp.dot(a0_ref[...], wa0_ref[...],
                        preferred_element_type=jnp.float32)
    acc = acc + jnp.dot(a1_ref[...], wa1_ref[...],
                        preferred_element_type=jnp.float32)
    acc = acc + b_ref[...]
    o_ref[...] = jnp.maximum(acc, 0.0)


_tc_dense = pl.pallas_call(
    _tc_body,
    grid=(N // RB,),
    in_specs=[
        pl.BlockSpec((RB, D), lambda i: (i, 0)),
        pl.BlockSpec((RB, DH), lambda i: (i, 0)),
        pl.BlockSpec((RB, DH), lambda i: (i, 0)),
        pl.BlockSpec((D, D), lambda i: (0, 0)),
        pl.BlockSpec((DH, D), lambda i: (0, 0)),
        pl.BlockSpec((DH, D), lambda i: (0, 0)),
        pl.BlockSpec((1, D), lambda i: (0, 0)),
    ],
    out_specs=pl.BlockSpec((RB, D), lambda i: (i, 0)),
    out_shape=jax.ShapeDtypeStruct((N, D), jnp.float32),
)


@functools.partial(
    pl.kernel,
    out_type=jax.ShapeDtypeStruct((4, NF, 32), jnp.float32),
    mesh=_mesh(),
    compiler_params=pltpu.CompilerParams(use_tc_tiling_on_sc=False),
    scratch_types=[
        pltpu.VMEM((CPT_C, CH), jnp.int32),
        pltpu.VMEM((CPT_C, CH), jnp.int32),
        pltpu.VMEM((CPT_C * CH,), jnp.float32),
        pltpu.VMEM((CH, 32), jnp.float32),
        pltpu.VMEM((CH, 32), jnp.float32),
        pltpu.VMEM((CH, 32), jnp.float32),
        pltpu.VMEM((CH, 32), jnp.float32),
        pltpu.VMEM((200, 32), jnp.float32),
        pltpu.VMEM_SHARED((NF, 32), jnp.float32),
        pltpu.SemaphoreType.DMA,
        pltpu.SemaphoreType.DMA,
        pltpu.SemaphoreType.DMA,
        pltpu.SemaphoreType.DMA,
    ],
)
def _sc_unpool(h_hbm, ps_hbm, fid_hbm, wp_hbm, out_hbm,
               gi_v, fi_v, wp_v, rows0_v, rows1_v, srows0_v, srows1_v,
               zero_v, acc, gsem0, gsem1, ssem0, ssem1):
    c = lax.axis_index("c")
    s = lax.axis_index("s")
    rows = (rows0_v, rows1_v)
    srows = (srows0_v, srows1_v)
    gsem = (gsem0, gsem1)
    ssem = (ssem0, ssem1)

    zf = jnp.zeros((L,), jnp.float32)

    def zrow(r, carry):
        zero_v[r, pl.ds(0, L)] = zf
        zero_v[r, pl.ds(L, L)] = zf
        return carry

    lax.fori_loop(0, 200, zrow, 0)
    nz = (NF // 200 - s + NS - 1) // NS

    def zchunk(i, carry):
        ch = s + i * NS
        pltpu.sync_copy(zero_v, acc.at[pl.ds(ch * 200, 200), :])
        return carry

    pltpu.sync_copy(fid_hbm.at[s], fi_v)
    pltpu.sync_copy(wp_hbm.at[s], wp_v)

    def gather(ci, b):
        return pltpu.async_copy(h_hbm.at[gi_v.at[ci]], rows[b], gsem[b])

    def scatter(ci, b):
        return pltpu.async_copy(srows[b], acc.at[fi_v.at[ci]], ssem[b],
                                add=True)

    for blk in range(2):
        cb = c * 2 + blk
        lax.fori_loop(0, nz, zchunk, 0)
        pltpu.sync_copy(ps_hbm.at[s], gi_v)
        offv = lax.broadcast(cb * N, (L,))

        def shift(r, carry):
            for k in range(CH // L):
                gi_v[r, pl.ds(k * L, L)] = gi_v[r, pl.ds(k * L, L)] + offv
            return carry

        lax.fori_loop(0, CPT_C, shift, 0)
        plsc.subcore_barrier()

        gather(0, 0)
        gather(1, 1)

        def pair(t, carry):
            for b in range(2):
                ci = 2 * t + b
                pltpu.make_async_copy(h_hbm.at[gi_v.at[ci]], rows[b],
                                      gsem[b]).wait()

                @pl.when(t > 0)
                def _drain():
                    pltpu.make_async_copy(srows[b], acc.at[fi_v.at[ci]],
                                          ssem[b]).wait()

                def subgroup(q, ecarry):
                    wq = wp_v[pl.ds(ci * CH + q * L, L)]
                    for j2 in range(L):
                        bw = lax.broadcast(wq[j2], (L,))
                        j = q * L + j2
                        srows[b][j, pl.ds(0, L)] = (
                            rows[b][j, pl.ds(0, L)] * bw)
                        srows[b][j, pl.ds(L, L)] = (
                            rows[b][j, pl.ds(L, L)] * bw)
                    return ecarry

                lax.fori_loop(0, CH // L, subgroup, 0)
                scatter(ci, b)

                @pl.when(t < (CPT_C // 2) - 1)
                def _prefetch():
                    gather(ci + 2, b)
            return carry

        lax.fori_loop(0, CPT_C // 2, pair, 0)
        for b in range(2):
            pltpu.make_async_copy(srows[b], acc.at[fi_v.at[CPT_C - 2 + b]],
                                  ssem[b]).wait()
        plsc.subcore_barrier()

        def fchunk(i, carry):
            ch = s + i * NS
            pltpu.sync_copy(acc.at[pl.ds(ch * 200, 200), :],
                            out_hbm.at[cb, pl.ds(ch * 200, 200), :])
            return carry

        lax.fori_loop(0, nz, fchunk, 0)
        plsc.subcore_barrier()


def kernel(x, edge_index, edge_attr, pool_edge_index, pool_edge_attr,
           n_fine, W_root, W_agg, b, we):
    pe = EPAD - E
    src = jnp.pad(edge_index[0], (0, pe)).reshape(NS, CPT_A, CH)
    dst = jnp.pad(edge_index[1], (0, pe)).reshape(NS, CPT_A, CH)
    ea = jnp.pad(edge_attr.T, ((0, 0), (0, pe))).reshape(4, NS, CPT_A * CH)
    we16 = jnp.zeros((16,), jnp.float32).at[:4].set(we[:, 0])
    xcs = x.reshape(N, NC, DH).transpose(1, 0, 2).reshape(NC * N, DH)

    a01 = _sc_aggr(xcs, src, dst, ea, we16)
    h = _tc_dense(x, a01[0], a01[1], W_root, W_agg[:DH], W_agg[DH:],
                  b.reshape(1, D))

    hcb = h.reshape(N, 4, 32).transpose(1, 0, 2).reshape(4 * N, 32)
    pp = EPPAD - EP
    ps = jnp.pad(pool_edge_index[1], (0, pp)).reshape(NS, CPT_C, CH)
    fid = jnp.minimum(pool_edge_index[0], n_fine - 1).astype(jnp.int32)
    fid = jnp.pad(fid, (0, pp)).reshape(NS, CPT_C, CH)
    wp = jnp.pad(pool_edge_attr[:, 0], (0, pp)).reshape(NS, CPT_C * CH)
    out_cb = _sc_unpool(hcb, ps, fid, wp)
    return out_cb.transpose(1, 0, 2).reshape(NF, D)


# speedup vs baseline: 3.5100x; 1.1757x over previous
import functools

import jax
import jax.numpy as jnp
from jax import lax
from jax.experimental import pallas as pl
from jax.experimental.pallas import tpu as pltpu
from jax.experimental.pallas import tpu_sc as plsc

N = 10000
D = 128
DH = 64
E = 320000
EP = 80000
NF = 40000
L = 16
NC = 2
NS = 16
NW = NC * NS

CH = 128
EPAD = 327680
CPT_A = EPAD // (NS * CH)
HB_A = CPT_A // 2
EPPAD = 81920
CPT_C = EPPAD // (NS * CH)


def _mesh():
    return plsc.VectorSubcoreMesh(core_axis_name="c", subcore_axis_name="s")


@functools.partial(
    pl.kernel,
    out_type=jax.ShapeDtypeStruct((NC, N, DH), jnp.float32),
    mesh=_mesh(),
    compiler_params=pltpu.CompilerParams(use_tc_tiling_on_sc=False),
    scratch_types=[
        pltpu.VMEM((HB_A, CH), jnp.int32),
        pltpu.VMEM((HB_A, CH), jnp.int32),
        pltpu.VMEM((CPT_A * CH,), jnp.float32),
        pltpu.VMEM((HB_A * CH // 4,), jnp.float32),
        pltpu.VMEM((CH, DH), jnp.float32),
        pltpu.VMEM((CH, DH), jnp.float32),
        pltpu.VMEM((CH, DH), jnp.float32),
        pltpu.VMEM((CH, DH), jnp.float32),
        pltpu.VMEM((40, DH), jnp.float32),
        pltpu.VMEM((16,), jnp.float32),
        pltpu.VMEM_SHARED((N, DH), jnp.float32),
        pltpu.SemaphoreType.DMA,
        pltpu.SemaphoreType.DMA,
        pltpu.SemaphoreType.DMA,
        pltpu.SemaphoreType.DMA,
    ],
)
def _sc_aggr(x_hbm, src_hbm, dst_hbm, ea_hbm, we_hbm, out_hbm,
             src_v, dst_v, w_v, tmp_v, rows0_v, rows1_v, srows0_v, srows1_v,
             zero_v, we_v, acc, gsem0, gsem1, ssem0, ssem1):
    c = lax.axis_index("c")
    s = lax.axis_index("s")
    rows = (rows0_v, rows1_v)
    srows = (srows0_v, srows1_v)
    gsem = (gsem0, gsem1)
    ssem = (ssem0, ssem1)

    zf = jnp.zeros((L,), jnp.float32)

    def zrow(r, carry):
        for k in range(DH // L):
            zero_v[r, pl.ds(k * L, L)] = zf
        return carry

    lax.fori_loop(0, 40, zrow, 0)

    def zchunk(i, carry):
        ch = s + i * NS
        pltpu.sync_copy(zero_v, acc.at[pl.ds(ch * 40, 40), :])
        return carry

    nz = (N // 40 - s + NS - 1) // NS
    lax.fori_loop(0, nz, zchunk, 0)

    pltpu.sync_copy(we_hbm, we_v)
    wev = we_v[...]
    QW = HB_A * CH // 4
    for k in range(4):
        for qb in range(8):
            pltpu.sync_copy(ea_hbm.at[k, s, pl.ds(qb * QW, QW)], tmp_v)

            def wacc(i, carry):
                o = qb * QW + i * L
                t = tmp_v[pl.ds(i * L, L)] * wev[k]
                if k == 0:
                    w_v[pl.ds(o, L)] = t
                else:
                    w_v[pl.ds(o, L)] = w_v[pl.ds(o, L)] + t
                return carry

            lax.fori_loop(0, QW // L, wacc, 0)
    plsc.subcore_barrier()

    coff = lax.broadcast(c * N, (L,))

    for hb in range(2):
        pltpu.sync_copy(src_hbm.at[s, pl.ds(hb * HB_A, HB_A), :], src_v)
        pltpu.sync_copy(dst_hbm.at[s, pl.ds(hb * HB_A, HB_A), :], dst_v)

        def shift(r, carry):
            for k in range(CH // L):
                src_v[r, pl.ds(k * L, L)] = src_v[r, pl.ds(k * L, L)] + coff
            return carry

        lax.fori_loop(0, HB_A, shift, 0)

        def gather(ci, b):
            return pltpu.async_copy(x_hbm.at[src_v.at[ci]], rows[b], gsem[b])

        def scatter(ci, b):
            return pltpu.async_copy(srows[b], acc.at[dst_v.at[ci]], ssem[b],
                                    add=True)

        gather(0, 0)
        gather(1, 1)
        wbase = hb * (HB_A * CH)

        def pair(t, carry):
            for b in range(2):
                ci = 2 * t + b
                pltpu.make_async_copy(x_hbm.at[src_v.at[ci]], rows[b],
                                      gsem[b]).wait()

                @pl.when(t > 0)
                def _drain():
                    pltpu.make_async_copy(srows[b], acc.at[dst_v.at[ci]],
                                          ssem[b]).wait()

                def subgroup(q, ecarry):
                    wq = w_v[pl.ds(wbase + ci * CH + q * L, L)]
                    for j2 in range(L):
                        bw = lax.broadcast(wq[j2], (L,))
                        j = q * L + j2
                        for k in range(DH // L):
                            srows[b][j, pl.ds(k * L, L)] = (
                                rows[b][j, pl.ds(k * L, L)] * bw)
                    return ecarry

                lax.fori_loop(0, CH // L, subgroup, 0)
                scatter(ci, b)

                @pl.when(t < HB_A // 2 - 1)
                def _prefetch():
                    gather(ci + 2, b)
            return carry

        lax.fori_loop(0, HB_A // 2, pair, 0)
        for b in range(2):
            pltpu.make_async_copy(srows[b], acc.at[dst_v.at[HB_A - 2 + b]],
                                  ssem[b]).wait()

    plsc.subcore_barrier()

    def fchunk(i, carry):
        ch = s + i * NS
        pltpu.sync_copy(acc.at[pl.ds(ch * 40, 40), :],
                        out_hbm.at[c, pl.ds(ch * 40, 40), :])
        return carry

    lax.fori_loop(0, nz, fchunk, 0)


RB = 1000


def _tc_body(x_ref, a0_ref, a1_ref, wr_ref, wa0_ref, wa1_ref, b_ref, o_ref):
    acc = jnp.dot(x_ref[...], wr_ref[...], preferred_element_type=jnp.float32)
    acc = acc + jnp.dot(a0_ref[...], wa0_ref[...],
                        preferred_element_type=jnp.float32)
    acc = acc + jnp.dot(a1_ref[...], wa1_ref[...],
                        preferred_element_type=jnp.float32)
    acc = acc + b_ref[...]
    o_ref[...] = jnp.maximum(acc, 0.0)


_tc_dense = pl.pallas_call(
    _tc_body,
    grid=(N // RB,),
    in_specs=[
        pl.BlockSpec((RB, D), lambda i: (i, 0)),
        pl.BlockSpec((RB, DH), lambda i: (i, 0)),
        pl.BlockSpec((RB, DH), lambda i: (i, 0)),
        pl.BlockSpec((D, D), lambda i: (0, 0)),
        pl.BlockSpec((DH, D), lambda i: (0, 0)),
        pl.BlockSpec((DH, D), lambda i: (0, 0)),
        pl.BlockSpec((1, D), lambda i: (0, 0)),
    ],
    out_specs=pl.BlockSpec((RB, D), lambda i: (i, 0)),
    out_shape=jax.ShapeDtypeStruct((N, D), jnp.float32),
)


@functools.partial(
    pl.kernel,
    out_type=jax.ShapeDtypeStruct((4, NF, 32), jnp.float32),
    mesh=_mesh(),
    compiler_params=pltpu.CompilerParams(use_tc_tiling_on_sc=False),
    scratch_types=[
        pltpu.VMEM((CPT_C, CH), jnp.int32),
        pltpu.VMEM((CPT_C, CH), jnp.int32),
        pltpu.VMEM((CPT_C * CH,), jnp.float32),
        pltpu.VMEM((CH, 32), jnp.float32),
        pltpu.VMEM((CH, 32), jnp.float32),
        pltpu.VMEM((CH, 32), jnp.float32),
        pltpu.VMEM((CH, 32), jnp.float32),
        pltpu.VMEM((200, 32), jnp.float32),
        pltpu.VMEM_SHARED((NF, 32), jnp.float32),
        pltpu.SemaphoreType.DMA,
        pltpu.SemaphoreType.DMA,
        pltpu.SemaphoreType.DMA,
        pltpu.SemaphoreType.DMA,
    ],
)
def _sc_unpool(h_hbm, ps_hbm, fid_hbm, wp_hbm, out_hbm,
               gi_v, fi_v, wp_v, rows0_v, rows1_v, srows0_v, srows1_v,
               zero_v, acc, gsem0, gsem1, ssem0, ssem1):
    c = lax.axis_index("c")
    s = lax.axis_index("s")
    rows = (rows0_v, rows1_v)
    srows = (srows0_v, srows1_v)
    gsem = (gsem0, gsem1)
    ssem = (ssem0, ssem1)

    zf = jnp.zeros((L,), jnp.float32)

    def zrow(r, carry):
        zero_v[r, pl.ds(0, L)] = zf
        zero_v[r, pl.ds(L, L)] = zf
        return carry

    lax.fori_loop(0, 200, zrow, 0)
    nz = (NF // 200 - s + NS - 1) // NS

    def zchunk(i, carry):
        ch = s + i * NS
        pltpu.sync_copy(zero_v, acc.at[pl.ds(ch * 200, 200), :])
        return carry

    pltpu.sync_copy(fid_hbm.at[s], fi_v)
    pltpu.sync_copy(wp_hbm.at[s], wp_v)

    def gather(ci, b):
        return pltpu.async_copy(h_hbm.at[gi_v.at[ci]], rows[b], gsem[b])

    def scatter(ci, b):
        return pltpu.async_copy(srows[b], acc.at[fi_v.at[ci]], ssem[b],
                                add=True)

    for blk in range(2):
        cb = c * 2 + blk
        lax.fori_loop(0, nz, zchunk, 0)
        pltpu.sync_copy(ps_hbm.at[s], gi_v)
        offv = lax.broadcast(cb * N, (L,))

        def shift(r, carry):
            for k in range(CH // L):
                gi_v[r, pl.ds(k * L, L)] = gi_v[r, pl.ds(k * L, L)] + offv
            return carry

        lax.fori_loop(0, CPT_C, shift, 0)
        plsc.subcore_barrier()

        gather(0, 0)
        gather(1, 1)

        def pair(t, carry):
            for b in range(2):
                ci = 2 * t + b
                pltpu.make_async_copy(h_hbm.at[gi_v.at[ci]], rows[b],
                                      gsem[b]).wait()

                @pl.when(t > 0)
                def _drain():
                    pltpu.make_async_copy(srows[b], acc.at[fi_v.at[ci]],
                                          ssem[b]).wait()

                def subgroup(q, ecarry):
                    wq = wp_v[pl.ds(ci * CH + q * L, L)]
                    for j2 in range(L):
                        bw = lax.broadcast(wq[j2], (L,))
                        j = q * L + j2
                        srows[b][j, pl.ds(0, L)] = (
                            rows[b][j, pl.ds(0, L)] * bw)
                        srows[b][j, pl.ds(L, L)] = (
                            rows[b][j, pl.ds(L, L)] * bw)
                    return ecarry

                lax.fori_loop(0, CH // L, subgroup, 0)
                scatter(ci, b)

                @pl.when(t < (CPT_C // 2) - 1)
                def _prefetch():
                    gather(ci + 2, b)
            return carry

        lax.fori_loop(0, CPT_C // 2, pair, 0)
        for b in range(2):
            pltpu.make_async_copy(srows[b], acc.at[fi_v.at[CPT_C - 2 + b]],
                                  ssem[b]).wait()
        plsc.subcore_barrier()

        def fchunk(i, carry):
            ch = s + i * NS
            pltpu.sync_copy(acc.at[pl.ds(ch * 200, 200), :],
                            out_hbm.at[cb, pl.ds(ch * 200, 200), :])
            return carry

        lax.fori_loop(0, nz, fchunk, 0)
        plsc.subcore_barrier()


def kernel(x, edge_index, edge_attr, pool_edge_index, pool_edge_attr,
           n_fine, W_root, W_agg, b, we):
    pe = EPAD - E
    src = jnp.pad(edge_index[0], (0, pe)).reshape(NS, CPT_A, CH)
    dst = jnp.pad(edge_index[1], (0, pe)).reshape(NS, CPT_A, CH)
    ea = jnp.pad(edge_attr.T, ((0, 0), (0, pe))).reshape(4, NS, CPT_A * CH)
    we16 = jnp.zeros((16,), jnp.float32).at[:4].set(we[:, 0])
    xcs = x.reshape(N, NC, DH).transpose(1, 0, 2).reshape(NC * N, DH)

    a01 = _sc_aggr(xcs, src, dst, ea, we16)
    h = _tc_dense(x, a01[0], a01[1], W_root, W_agg[:DH], W_agg[DH:],
                  b.reshape(1, D))

    hcb = h.reshape(N, 4, 32).transpose(1, 0, 2).reshape(4 * N, 32)
    pp = EPPAD - EP
    ps = jnp.pad(pool_edge_index[1], (0, pp)).reshape(NS, CPT_C, CH)
    fid = jnp.minimum(pool_edge_index[0], n_fine - 1).astype(jnp.int32)
    fid = jnp.pad(fid, (0, pp)).reshape(NS, CPT_C, CH)
    wp = jnp.pad(pool_edge_attr[:, 0], (0, pp)).reshape(NS, CPT_C * CH)
    out_cb = _sc_unpool(hcb, ps, fid, wp)
    return out_cb.transpose(1, 0, 2).reshape(NF, D)


# speedup vs baseline: 3.6980x; 1.0536x over previous
import functools

import jax
import jax.numpy as jnp
from jax import lax
from jax.experimental import pallas as pl
from jax.experimental.pallas import tpu as pltpu
from jax.experimental.pallas import tpu_sc as plsc

N = 10000
D = 128
DH = 64
E = 320000
EP = 80000
NF = 40000
L = 16
NC = 2
NS = 16
NW = NC * NS

CH = 128
EPAD = 327680
CPT_A = EPAD // (NS * CH)
HB_A = CPT_A // 2
EPPAD = 81920
CPT_C = EPPAD // (NS * CH)


def _mesh():
    return plsc.VectorSubcoreMesh(core_axis_name="c", subcore_axis_name="s")


@functools.partial(
    pl.kernel,
    out_type=jax.ShapeDtypeStruct((NC, N, DH), jnp.float32),
    mesh=_mesh(),
    compiler_params=pltpu.CompilerParams(use_tc_tiling_on_sc=False),
    scratch_types=[
        pltpu.VMEM((HB_A, CH), jnp.int32),
        pltpu.VMEM((HB_A, CH), jnp.int32),
        pltpu.VMEM((CPT_A * CH,), jnp.float32),
        pltpu.VMEM((HB_A * CH // 4,), jnp.float32),
        pltpu.VMEM((CH, DH), jnp.float32),
        pltpu.VMEM((CH, DH), jnp.float32),
        pltpu.VMEM((CH, DH), jnp.float32),
        pltpu.VMEM((CH, DH), jnp.float32),
        pltpu.VMEM((40, DH), jnp.float32),
        pltpu.VMEM((16,), jnp.float32),
        pltpu.VMEM_SHARED((N, DH), jnp.float32),
        pltpu.SemaphoreType.DMA,
        pltpu.SemaphoreType.DMA,
        pltpu.SemaphoreType.DMA,
        pltpu.SemaphoreType.DMA,
    ],
)
def _sc_aggr(x_hbm, src_hbm, dst_hbm, ea_hbm, we_hbm, out_hbm,
             src_v, dst_v, w_v, tmp_v, rows0_v, rows1_v, srows0_v, srows1_v,
             zero_v, we_v, acc, gsem0, gsem1, ssem0, ssem1):
    c = lax.axis_index("c")
    s = lax.axis_index("s")
    rows = (rows0_v, rows1_v)
    srows = (srows0_v, srows1_v)
    gsem = (gsem0, gsem1)
    ssem = (ssem0, ssem1)

    zf = jnp.zeros((L,), jnp.float32)

    def zrow(r, carry):
        for k in range(DH // L):
            zero_v[r, pl.ds(k * L, L)] = zf
        return carry

    lax.fori_loop(0, 40, zrow, 0)

    def zchunk(i, carry):
        ch = s + i * NS
        pltpu.sync_copy(zero_v, acc.at[pl.ds(ch * 40, 40), :])
        return carry

    nz = (N // 40 - s + NS - 1) // NS
    lax.fori_loop(0, nz, zchunk, 0)

    pltpu.sync_copy(we_hbm, we_v)
    wev = we_v[...]
    QW = HB_A * CH // 4
    for k in range(4):
        for qb in range(8):
            pltpu.sync_copy(ea_hbm.at[k, s, pl.ds(qb * QW, QW)], tmp_v)

            def wacc(i, carry):
                o = qb * QW + i * L
                t = tmp_v[pl.ds(i * L, L)] * wev[k]
                if k == 0:
                    w_v[pl.ds(o, L)] = t
                else:
                    w_v[pl.ds(o, L)] = w_v[pl.ds(o, L)] + t
                return carry

            lax.fori_loop(0, QW // L, wacc, 0)
    plsc.subcore_barrier()

    coff = lax.broadcast(c, (L,))

    for hb in range(2):
        pltpu.sync_copy(src_hbm.at[s, pl.ds(hb * HB_A, HB_A), :], src_v)
        pltpu.sync_copy(dst_hbm.at[s, pl.ds(hb * HB_A, HB_A), :], dst_v)

        def shift(r, carry):
            for k in range(CH // L):
                src_v[r, pl.ds(k * L, L)] = (
                    src_v[r, pl.ds(k * L, L)] * 2 + coff)
            return carry

        lax.fori_loop(0, HB_A, shift, 0)

        def gather(ci, b):
            return pltpu.async_copy(x_hbm.at[src_v.at[ci]], rows[b], gsem[b])

        def scatter(ci, b):
            return pltpu.async_copy(srows[b], acc.at[dst_v.at[ci]], ssem[b],
                                    add=True)

        gather(0, 0)
        gather(1, 1)
        wbase = hb * (HB_A * CH)

        def pair(t, carry):
            for b in range(2):
                ci = 2 * t + b
                pltpu.make_async_copy(x_hbm.at[src_v.at[ci]], rows[b],
                                      gsem[b]).wait()

                @pl.when(t > 0)
                def _drain():
                    pltpu.make_async_copy(srows[b], acc.at[dst_v.at[ci]],
                                          ssem[b]).wait()

                def subgroup(q, ecarry):
                    wq = w_v[pl.ds(wbase + ci * CH + q * L, L)]
                    for j2 in range(L):
                        bw = lax.broadcast(wq[j2], (L,))
                        j = q * L + j2
                        for k in range(DH // L):
                            srows[b][j, pl.ds(k * L, L)] = (
                                rows[b][j, pl.ds(k * L, L)] * bw)
                    return ecarry

                lax.fori_loop(0, CH // L, subgroup, 0)
                scatter(ci, b)

                @pl.when(t < HB_A // 2 - 1)
                def _prefetch():
                    gather(ci + 2, b)
            return carry

        lax.fori_loop(0, HB_A // 2, pair, 0)
        for b in range(2):
            pltpu.make_async_copy(srows[b], acc.at[dst_v.at[HB_A - 2 + b]],
                                  ssem[b]).wait()

    plsc.subcore_barrier()

    def fchunk(i, carry):
        ch = s + i * NS
        pltpu.sync_copy(acc.at[pl.ds(ch * 40, 40), :],
                        out_hbm.at[c, pl.ds(ch * 40, 40), :])
        return carry

    lax.fori_loop(0, nz, fchunk, 0)


RB = 1000


def _tc_body(x_ref, a0_ref, a1_ref, wr_ref, wa0_ref, wa1_ref, b_ref, o_ref):
    acc = jnp.dot(x_ref[...], wr_ref[...], preferred_element_type=jnp.float32)
    acc = acc + jnp.dot(a0_ref[...], wa0_ref[...],
                        preferred_element_type=jnp.float32)
    acc = acc + jnp.dot(a1_ref[...], wa1_ref[...],
                        preferred_element_type=jnp.float32)
    acc = acc + b_ref[...]
    o_ref[...] = jnp.maximum(acc, 0.0)


_tc_dense = pl.pallas_call(
    _tc_body,
    grid=(N // RB,),
    in_specs=[
        pl.BlockSpec((RB, D), lambda i: (i, 0)),
        pl.BlockSpec((RB, DH), lambda i: (i, 0)),
        pl.BlockSpec((RB, DH), lambda i: (i, 0)),
        pl.BlockSpec((D, D), lambda i: (0, 0)),
        pl.BlockSpec((DH, D), lambda i: (0, 0)),
        pl.BlockSpec((DH, D), lambda i: (0, 0)),
        pl.BlockSpec((1, D), lambda i: (0, 0)),
    ],
    out_specs=pl.BlockSpec((RB, D), lambda i: (i, 0)),
    out_shape=jax.ShapeDtypeStruct((N, D), jnp.float32),
)


@functools.partial(
    pl.kernel,
    out_type=jax.ShapeDtypeStruct((NF, D), jnp.float32),
    mesh=_mesh(),
    compiler_params=pltpu.CompilerParams(use_tc_tiling_on_sc=False),
    scratch_types=[
        pltpu.VMEM((CPT_C, CH), jnp.int32),
        pltpu.VMEM((CPT_C, CH), jnp.int32),
        pltpu.VMEM((CPT_C * CH,), jnp.float32),
        pltpu.VMEM((CH, 32), jnp.float32),
        pltpu.VMEM((CH, 32), jnp.float32),
        pltpu.VMEM((CH, 32), jnp.float32),
        pltpu.VMEM((CH, 32), jnp.float32),
        pltpu.VMEM((200, 32), jnp.float32),
        pltpu.VMEM_SHARED((NF, 32), jnp.float32),
        pltpu.SemaphoreType.DMA,
        pltpu.SemaphoreType.DMA,
        pltpu.SemaphoreType.DMA,
        pltpu.SemaphoreType.DMA,
    ],
)
def _sc_unpool(h_hbm, ps_hbm, fid_hbm, wp_hbm, out_hbm,
               gi_v, fi_v, wp_v, rows0_v, rows1_v, srows0_v, srows1_v,
               zero_v, acc, gsem0, gsem1, ssem0, ssem1):
    c = lax.axis_index("c")
    s = lax.axis_index("s")
    rows = (rows0_v, rows1_v)
    srows = (srows0_v, srows1_v)
    gsem = (gsem0, gsem1)
    ssem = (ssem0, ssem1)

    zf = jnp.zeros((L,), jnp.float32)

    def zrow(r, carry):
        zero_v[r, pl.ds(0, L)] = zf
        zero_v[r, pl.ds(L, L)] = zf
        return carry

    lax.fori_loop(0, 200, zrow, 0)
    nz = (NF // 200 - s + NS - 1) // NS

    def zchunk(i, carry):
        ch = s + i * NS
        pltpu.sync_copy(zero_v, acc.at[pl.ds(ch * 200, 200), :])
        return carry

    pltpu.sync_copy(fid_hbm.at[s], fi_v)
    pltpu.sync_copy(wp_hbm.at[s], wp_v)

    def gather(ci, b):
        return pltpu.async_copy(h_hbm.at[gi_v.at[ci]], rows[b], gsem[b])

    def scatter(ci, b):
        return pltpu.async_copy(srows[b], acc.at[fi_v.at[ci]], ssem[b],
                                add=True)

    for blk in range(2):
        cb = c * 2 + blk
        lax.fori_loop(0, nz, zchunk, 0)
        pltpu.sync_copy(ps_hbm.at[s], gi_v)
        offv = lax.broadcast(cb, (L,))

        def shift(r, carry):
            for k in range(CH // L):
                gi_v[r, pl.ds(k * L, L)] = (
                    gi_v[r, pl.ds(k * L, L)] * 4 + offv)
            return carry

        lax.fori_loop(0, CPT_C, shift, 0)
        plsc.subcore_barrier()

        gather(0, 0)
        gather(1, 1)

        def pair(t, carry):
            for b in range(2):
                ci = 2 * t + b
                pltpu.make_async_copy(h_hbm.at[gi_v.at[ci]], rows[b],
                                      gsem[b]).wait()

                @pl.when(t > 0)
                def _drain():
                    pltpu.make_async_copy(srows[b], acc.at[fi_v.at[ci]],
                                          ssem[b]).wait()

                def subgroup(q, ecarry):
                    wq = wp_v[pl.ds(ci * CH + q * L, L)]
                    for j2 in range(L):
                        bw = lax.broadcast(wq[j2], (L,))
                        j = q * L + j2
                        srows[b][j, pl.ds(0, L)] = (
                            rows[b][j, pl.ds(0, L)] * bw)
                        srows[b][j, pl.ds(L, L)] = (
                            rows[b][j, pl.ds(L, L)] * bw)
                    return ecarry

                lax.fori_loop(0, CH // L, subgroup, 0)
                scatter(ci, b)

                @pl.when(t < (CPT_C // 2) - 1)
                def _prefetch():
                    gather(ci + 2, b)
            return carry

        lax.fori_loop(0, CPT_C // 2, pair, 0)
        for b in range(2):
            pltpu.make_async_copy(srows[b], acc.at[fi_v.at[CPT_C - 2 + b]],
                                  ssem[b]).wait()
        plsc.subcore_barrier()

        def fchunk(i, carry):
            ch = s + i * NS
            pltpu.sync_copy(acc.at[pl.ds(ch * 200, 200), :],
                            out_hbm.at[pl.ds(ch * 200, 200),
                                       pl.ds(cb * 32, 32)])
            return carry

        lax.fori_loop(0, nz, fchunk, 0)
        plsc.subcore_barrier()


def kernel(x, edge_index, edge_attr, pool_edge_index, pool_edge_attr,
           n_fine, W_root, W_agg, b, we):
    pe = EPAD - E
    src = jnp.pad(edge_index[0], (0, pe)).reshape(NS, CPT_A, CH)
    dst = jnp.pad(edge_index[1], (0, pe)).reshape(NS, CPT_A, CH)
    ea = jnp.pad(edge_attr.T, ((0, 0), (0, pe))).reshape(4, NS, CPT_A * CH)
    we16 = jnp.zeros((16,), jnp.float32).at[:4].set(we[:, 0])
    xcs = x.reshape(NC * N, DH)

    a01 = _sc_aggr(xcs, src, dst, ea, we16)
    h = _tc_dense(x, a01[0], a01[1], W_root, W_agg[:DH], W_agg[DH:],
                  b.reshape(1, D))

    hcb = h.reshape(4 * N, 32)
    pp = EPPAD - EP
    ps = jnp.pad(pool_edge_index[1], (0, pp)).reshape(NS, CPT_C, CH)
    fid = jnp.minimum(pool_edge_index[0], n_fine - 1).astype(jnp.int32)
    fid = jnp.pad(fid, (0, pp)).reshape(NS, CPT_C, CH)
    wp = jnp.pad(pool_edge_attr[:, 0], (0, pp)).reshape(NS, CPT_C * CH)
    return _sc_unpool(hcb, ps, fid, wp)


# speedup vs baseline: 4.1373x; 1.1188x over previous
import functools

import jax
import jax.numpy as jnp
from jax import lax
from jax.experimental import pallas as pl
from jax.experimental.pallas import tpu as pltpu
from jax.experimental.pallas import tpu_sc as plsc

N = 10000
D = 128
DH = 64
E = 320000
EP = 80000
NF = 40000
L = 16
NC = 2
NS = 16
NW = NC * NS

CH = 128
EPAD = 327680
CPT_A = EPAD // (NS * CH)
HB_A = CPT_A // 2
EPPAD = 81920
CPT_C = EPPAD // (NS * CH)


def _mesh():
    return plsc.VectorSubcoreMesh(core_axis_name="c", subcore_axis_name="s")


@functools.partial(
    pl.kernel,
    out_type=jax.ShapeDtypeStruct((NC, N, DH), jnp.float32),
    mesh=_mesh(),
    compiler_params=pltpu.CompilerParams(use_tc_tiling_on_sc=False,
                                         needs_layout_passes=False),
    scratch_types=[
        pltpu.VMEM((HB_A, CH), jnp.int32),
        pltpu.VMEM((HB_A, CH), jnp.int32),
        pltpu.VMEM((CPT_A * CH,), jnp.float32),
        pltpu.VMEM((HB_A * CH // 4,), jnp.float32),
        pltpu.VMEM((CH, DH), jnp.bfloat16),
        pltpu.VMEM((CH, DH), jnp.bfloat16),
        pltpu.VMEM((CH, DH), jnp.float32),
        pltpu.VMEM((CH, DH), jnp.float32),
        pltpu.VMEM((40, DH), jnp.float32),
        pltpu.VMEM((16,), jnp.float32),
        pltpu.VMEM_SHARED((N, DH), jnp.float32),
        pltpu.SemaphoreType.DMA,
        pltpu.SemaphoreType.DMA,
        pltpu.SemaphoreType.DMA,
        pltpu.SemaphoreType.DMA,
    ],
)
def _sc_aggr(x_hbm, src_hbm, dst_hbm, ea_hbm, we_hbm, out_hbm,
             src_v, dst_v, w_v, tmp_v, rows0_v, rows1_v, srows0_v, srows1_v,
             zero_v, we_v, acc, gsem0, gsem1, ssem0, ssem1):
    c = lax.axis_index("c")
    s = lax.axis_index("s")
    rows = (rows0_v, rows1_v)
    srows = (srows0_v, srows1_v)
    gsem = (gsem0, gsem1)
    ssem = (ssem0, ssem1)

    zf = jnp.zeros((L,), jnp.float32)

    def zrow(r, carry):
        for k in range(DH // L):
            zero_v[r, pl.ds(k * L, L)] = zf
        return carry

    lax.fori_loop(0, 40, zrow, 0)

    def zchunk(i, carry):
        ch = s + i * NS
        pltpu.sync_copy(zero_v, acc.at[pl.ds(ch * 40, 40), :])
        return carry

    nz = (N // 40 - s + NS - 1) // NS
    lax.fori_loop(0, nz, zchunk, 0)

    pltpu.sync_copy(we_hbm, we_v)
    wev = we_v[...]
    QW = HB_A * CH // 4
    for k in range(4):
        for qb in range(8):
            pltpu.sync_copy(ea_hbm.at[k, s, pl.ds(qb * QW, QW)], tmp_v)

            def wacc(i, carry):
                o = qb * QW + i * L
                t = tmp_v[pl.ds(i * L, L)] * wev[k]
                if k == 0:
                    w_v[pl.ds(o, L)] = t
                else:
                    w_v[pl.ds(o, L)] = w_v[pl.ds(o, L)] + t
                return carry

            lax.fori_loop(0, QW // L, wacc, 0)
    plsc.subcore_barrier()

    coff = lax.broadcast(c, (L,))

    for hb in range(2):
        pltpu.sync_copy(src_hbm.at[s, pl.ds(hb * HB_A, HB_A), :], src_v)
        pltpu.sync_copy(dst_hbm.at[s, pl.ds(hb * HB_A, HB_A), :], dst_v)

        def shift(r, carry):
            for k in range(CH // L):
                src_v[r, pl.ds(k * L, L)] = (
                    src_v[r, pl.ds(k * L, L)] * 2 + coff)
            return carry

        lax.fori_loop(0, HB_A, shift, 0)

        def gather(ci, b):
            return pltpu.async_copy(x_hbm.at[src_v.at[ci]], rows[b], gsem[b])

        def scatter(ci, b):
            return pltpu.async_copy(srows[b], acc.at[dst_v.at[ci]], ssem[b],
                                    add=True)

        gather(0, 0)
        gather(1, 1)
        wbase = hb * (HB_A * CH)

        def pair(t, carry):
            for b in range(2):
                ci = 2 * t + b
                pltpu.make_async_copy(x_hbm.at[src_v.at[ci]], rows[b],
                                      gsem[b]).wait()

                @pl.when(t > 0)
                def _drain():
                    pltpu.make_async_copy(srows[b], acc.at[dst_v.at[ci]],
                                          ssem[b]).wait()

                def subgroup(q, ecarry):
                    wq = w_v[pl.ds(wbase + ci * CH + q * L, L)]
                    for j2 in range(L):
                        bw = lax.broadcast(wq[j2], (L,))
                        j = q * L + j2
                        for k in range(DH // (2 * L)):
                            p = rows[b][j, pl.ds(k * 2 * L, 2 * L)]
                            ae, ao = plsc.unpack(
                                p, format=plsc.PackFormat.INTERLEAVED)
                            srows[b][j, pl.ds(k * 2 * L, L)] = ae * bw
                            srows[b][j, pl.ds(k * 2 * L + L, L)] = ao * bw
                    return ecarry

                lax.fori_loop(0, CH // L, subgroup, 0)
                scatter(ci, b)

                @pl.when(t < HB_A // 2 - 1)
                def _prefetch():
                    gather(ci + 2, b)
            return carry

        lax.fori_loop(0, HB_A // 2, pair, 0)
        for b in range(2):
            pltpu.make_async_copy(srows[b], acc.at[dst_v.at[HB_A - 2 + b]],
                                  ssem[b]).wait()

    plsc.subcore_barrier()

    def fchunk(i, carry):
        ch = s + i * NS
        pltpu.sync_copy(acc.at[pl.ds(ch * 40, 40), :],
                        out_hbm.at[c, pl.ds(ch * 40, 40), :])
        return carry

    lax.fori_loop(0, nz, fchunk, 0)


RB = 1000


def _tc_body(x_ref, a0_ref, a1_ref, wr_ref, wa0_ref, wa1_ref, b_ref, o_ref):
    acc = jnp.dot(x_ref[...], wr_ref[...], preferred_element_type=jnp.float32)
    acc = acc + jnp.dot(a0_ref[...], wa0_ref[...],
                        preferred_element_type=jnp.float32)
    acc = acc + jnp.dot(a1_ref[...], wa1_ref[...],
                        preferred_element_type=jnp.float32)
    acc = acc + b_ref[...]
    o_ref[...] = jnp.maximum(acc, 0.0)


_tc_dense = pl.pallas_call(
    _tc_body,
    grid=(N // RB,),
    in_specs=[
        pl.BlockSpec((RB, D), lambda i: (i, 0)),
        pl.BlockSpec((RB, DH), lambda i: (i, 0)),
        pl.BlockSpec((RB, DH), lambda i: (i, 0)),
        pl.BlockSpec((D, D), lambda i: (0, 0)),
        pl.BlockSpec((DH, D), lambda i: (0, 0)),
        pl.BlockSpec((DH, D), lambda i: (0, 0)),
        pl.BlockSpec((1, D), lambda i: (0, 0)),
    ],
    out_specs=pl.BlockSpec((RB, D), lambda i: (i, 0)),
    out_shape=jax.ShapeDtypeStruct((N, D), jnp.float32),
)


@functools.partial(
    pl.kernel,
    out_type=jax.ShapeDtypeStruct((NF, D), jnp.float32),
    mesh=_mesh(),
    compiler_params=pltpu.CompilerParams(use_tc_tiling_on_sc=False),
    scratch_types=[
        pltpu.VMEM((CPT_C, CH), jnp.int32),
        pltpu.VMEM((CPT_C, CH), jnp.int32),
        pltpu.VMEM((CPT_C * CH,), jnp.float32),
        pltpu.VMEM((CH, 32), jnp.float32),
        pltpu.VMEM((CH, 32), jnp.float32),
        pltpu.VMEM((CH, 32), jnp.float32),
        pltpu.VMEM((CH, 32), jnp.float32),
        pltpu.VMEM((200, 32), jnp.float32),
        pltpu.VMEM_SHARED((NF, 32), jnp.float32),
        pltpu.SemaphoreType.DMA,
        pltpu.SemaphoreType.DMA,
        pltpu.SemaphoreType.DMA,
        pltpu.SemaphoreType.DMA,
    ],
)
def _sc_unpool(h_hbm, ps_hbm, fid_hbm, wp_hbm, out_hbm,
               gi_v, fi_v, wp_v, rows0_v, rows1_v, srows0_v, srows1_v,
               zero_v, acc, gsem0, gsem1, ssem0, ssem1):
    c = lax.axis_index("c")
    s = lax.axis_index("s")
    rows = (rows0_v, rows1_v)
    srows = (srows0_v, srows1_v)
    gsem = (gsem0, gsem1)
    ssem = (ssem0, ssem1)

    zf = jnp.zeros((L,), jnp.float32)

    def zrow(r, carry):
        zero_v[r, pl.ds(0, L)] = zf
        zero_v[r, pl.ds(L, L)] = zf
        return carry

    lax.fori_loop(0, 200, zrow, 0)
    nz = (NF // 200 - s + NS - 1) // NS

    def zchunk(i, carry):
        ch = s + i * NS
        pltpu.sync_copy(zero_v, acc.at[pl.ds(ch * 200, 200), :])
        return carry

    pltpu.sync_copy(fid_hbm.at[s], fi_v)
    pltpu.sync_copy(wp_hbm.at[s], wp_v)

    def gather(ci, b):
        return pltpu.async_copy(h_hbm.at[gi_v.at[ci]], rows[b], gsem[b])

    def scatter(ci, b):
        return pltpu.async_copy(srows[b], acc.at[fi_v.at[ci]], ssem[b],
                                add=True)

    for blk in range(2):
        cb = c * 2 + blk
        lax.fori_loop(0, nz, zchunk, 0)
        pltpu.sync_copy(ps_hbm.at[s], gi_v)
        offv = lax.broadcast(cb, (L,))

        def shift(r, carry):
            for k in range(CH // L):
                gi_v[r, pl.ds(k * L, L)] = (
                    gi_v[r, pl.ds(k * L, L)] * 4 + offv)
            return carry

        lax.fori_loop(0, CPT_C, shift, 0)
        plsc.subcore_barrier()

        gather(0, 0)
        gather(1, 1)

        def pair(t, carry):
            for b in range(2):
                ci = 2 * t + b
                pltpu.make_async_copy(h_hbm.at[gi_v.at[ci]], rows[b],
                                      gsem[b]).wait()

                @pl.when(t > 0)
                def _drain():
                    pltpu.make_async_copy(srows[b], acc.at[fi_v.at[ci]],
                                          ssem[b]).wait()

                def subgroup(q, ecarry):
                    wq = wp_v[pl.ds(ci * CH + q * L, L)]
                    for j2 in range(L):
                        bw = lax.broadcast(wq[j2], (L,))
                        j = q * L + j2
                        srows[b][j, pl.ds(0, L)] = (
                            rows[b][j, pl.ds(0, L)] * bw)
                        srows[b][j, pl.ds(L, L)] = (
                            rows[b][j, pl.ds(L, L)] * bw)
                    return ecarry

                lax.fori_loop(0, CH // L, subgroup, 0)
                scatter(ci, b)

                @pl.when(t < (CPT_C // 2) - 1)
                def _prefetch():
                    gather(ci + 2, b)
            return carry

        lax.fori_loop(0, CPT_C // 2, pair, 0)
        for b in range(2):
            pltpu.make_async_copy(srows[b], acc.at[fi_v.at[CPT_C - 2 + b]],
                                  ssem[b]).wait()
        plsc.subcore_barrier()

        def fchunk(i, carry):
            ch = s + i * NS
            pltpu.sync_copy(acc.at[pl.ds(ch * 200, 200), :],
                            out_hbm.at[pl.ds(ch * 200, 200),
                                       pl.ds(cb * 32, 32)])
            return carry

        lax.fori_loop(0, nz, fchunk, 0)
        plsc.subcore_barrier()


def kernel(x, edge_index, edge_attr, pool_edge_index, pool_edge_attr,
           n_fine, W_root, W_agg, b, we):
    pe = EPAD - E
    src = jnp.pad(edge_index[0], (0, pe)).reshape(NS, CPT_A, CH)
    dst = jnp.pad(edge_index[1], (0, pe)).reshape(NS, CPT_A, CH)
    ea = jnp.pad(edge_attr.T, ((0, 0), (0, pe))).reshape(4, NS, CPT_A * CH)
    we16 = jnp.zeros((16,), jnp.float32).at[:4].set(we[:, 0])
    xcs = x.astype(jnp.bfloat16).reshape(NC * N, DH)

    a01 = _sc_aggr(xcs, src, dst, ea, we16)
    perm64 = jnp.array(
        [blk * 32 + (2 * r if r < 16 else 2 * (r - 16) + 1)
         for blk in range(2) for r in range(32)], dtype=jnp.int32)
    h = _tc_dense(x, a01[0], a01[1], W_root, W_agg[perm64],
                  W_agg[DH + perm64], b.reshape(1, D))

    hcb = h.reshape(4 * N, 32)
    pp = EPPAD - EP
    ps = jnp.pad(pool_edge_index[1], (0, pp)).reshape(NS, CPT_C, CH)
    fid = jnp.minimum(pool_edge_index[0], n_fine - 1).astype(jnp.int32)
    fid = jnp.pad(fid, (0, pp)).reshape(NS, CPT_C, CH)
    wp = jnp.pad(pool_edge_attr[:, 0], (0, pp)).reshape(NS, CPT_C * CH)
    return _sc_unpool(hcb, ps, fid, wp)


# speedup vs baseline: 4.2198x; 1.0199x over previous
import functools

import jax
import jax.numpy as jnp
from jax import lax
from jax.experimental import pallas as pl
from jax.experimental.pallas import tpu as pltpu
from jax.experimental.pallas import tpu_sc as plsc

N = 10000
D = 128
DH = 64
E = 320000
EP = 80000
NF = 40000
L = 16
NC = 2
NS = 16
NW = NC * NS

CH = 128
EPAD = 327680
CPT_A = EPAD // (NS * CH)
HB_A = CPT_A // 2
EPPAD = 81920
CPT_C = EPPAD // (NS * CH)


def _mesh():
    return plsc.VectorSubcoreMesh(core_axis_name="c", subcore_axis_name="s")


@functools.partial(
    pl.kernel,
    out_type=jax.ShapeDtypeStruct((NC, N, DH), jnp.float32),
    mesh=_mesh(),
    compiler_params=pltpu.CompilerParams(use_tc_tiling_on_sc=False,
                                         needs_layout_passes=False),
    scratch_types=[
        pltpu.VMEM((HB_A, CH), jnp.int32),
        pltpu.VMEM((HB_A, CH), jnp.int32),
        pltpu.VMEM((CPT_A * CH,), jnp.float32),
        pltpu.VMEM((HB_A * CH // 4,), jnp.float32),
        pltpu.VMEM((CH, DH), jnp.bfloat16),
        pltpu.VMEM((CH, DH), jnp.bfloat16),
        pltpu.VMEM((CH, DH), jnp.float32),
        pltpu.VMEM((CH, DH), jnp.float32),
        pltpu.VMEM((40, DH), jnp.float32),
        pltpu.VMEM((16,), jnp.float32),
        pltpu.VMEM_SHARED((N, DH), jnp.float32),
        pltpu.SemaphoreType.DMA,
        pltpu.SemaphoreType.DMA,
        pltpu.SemaphoreType.DMA,
        pltpu.SemaphoreType.DMA,
    ],
)
def _sc_aggr(x_hbm, src_hbm, dst_hbm, ea_hbm, we_hbm, out_hbm,
             src_v, dst_v, w_v, tmp_v, rows0_v, rows1_v, srows0_v, srows1_v,
             zero_v, we_v, acc, gsem0, gsem1, ssem0, ssem1):
    c = lax.axis_index("c")
    s = lax.axis_index("s")
    rows = (rows0_v, rows1_v)
    srows = (srows0_v, srows1_v)
    gsem = (gsem0, gsem1)
    ssem = (ssem0, ssem1)

    zf = jnp.zeros((L,), jnp.float32)

    def zrow(r, carry):
        for k in range(DH // L):
            zero_v[r, pl.ds(k * L, L)] = zf
        return carry

    lax.fori_loop(0, 40, zrow, 0)

    def zchunk(i, carry):
        ch = s + i * NS
        pltpu.sync_copy(zero_v, acc.at[pl.ds(ch * 40, 40), :])
        return carry

    nz = (N // 40 - s + NS - 1) // NS
    lax.fori_loop(0, nz, zchunk, 0)

    pltpu.sync_copy(we_hbm, we_v)
    wev = we_v[...]
    QW = HB_A * CH // 4
    for k in range(4):
        for qb in range(8):
            pltpu.sync_copy(ea_hbm.at[k, s, pl.ds(qb * QW, QW)], tmp_v)

            def wacc(i, carry):
                o = qb * QW + i * L
                t = tmp_v[pl.ds(i * L, L)] * wev[k]
                if k == 0:
                    w_v[pl.ds(o, L)] = t
                else:
                    w_v[pl.ds(o, L)] = w_v[pl.ds(o, L)] + t
                return carry

            lax.fori_loop(0, QW // L, wacc, 0)
    plsc.subcore_barrier()

    coff = lax.broadcast(c, (L,))

    for hb in range(2):
        pltpu.sync_copy(src_hbm.at[s, pl.ds(hb * HB_A, HB_A), :], src_v)
        pltpu.sync_copy(dst_hbm.at[s, pl.ds(hb * HB_A, HB_A), :], dst_v)

        def shift(r, carry):
            for k in range(CH // L):
                src_v[r, pl.ds(k * L, L)] = (
                    src_v[r, pl.ds(k * L, L)] * 2 + coff)
            return carry

        lax.fori_loop(0, HB_A, shift, 0)

        def gather(ci, b):
            return pltpu.async_copy(x_hbm.at[src_v.at[ci]], rows[b], gsem[b])

        def scatter(ci, b):
            return pltpu.async_copy(srows[b], acc.at[dst_v.at[ci]], ssem[b],
                                    add=True)

        gather(0, 0)
        gather(1, 1)
        wbase = hb * (HB_A * CH)

        def pair(t, carry):
            for b in range(2):
                ci = 2 * t + b
                pltpu.make_async_copy(x_hbm.at[src_v.at[ci]], rows[b],
                                      gsem[b]).wait()

                @pl.when(t > 0)
                def _drain():
                    pltpu.make_async_copy(srows[b], acc.at[dst_v.at[ci]],
                                          ssem[b]).wait()

                def subgroup(q, ecarry):
                    wq = w_v[pl.ds(wbase + ci * CH + q * L, L)]
                    for j2 in range(L):
                        bw = lax.broadcast(wq[j2], (L,))
                        j = q * L + j2
                        for k in range(DH // (2 * L)):
                            p = rows[b][j, pl.ds(k * 2 * L, 2 * L)]
                            ae, ao = plsc.unpack(
                                p, format=plsc.PackFormat.INTERLEAVED)
                            srows[b][j, pl.ds(k * 2 * L, L)] = ae * bw
                            srows[b][j, pl.ds(k * 2 * L + L, L)] = ao * bw
                    return ecarry

                lax.fori_loop(0, CH // L, subgroup, 0)
                scatter(ci, b)

                @pl.when(t < HB_A // 2 - 1)
                def _prefetch():
                    gather(ci + 2, b)
            return carry

        lax.fori_loop(0, HB_A // 2, pair, 0)
        for b in range(2):
            pltpu.make_async_copy(srows[b], acc.at[dst_v.at[HB_A - 2 + b]],
                                  ssem[b]).wait()

    plsc.subcore_barrier()

    def fchunk(i, carry):
        ch = s + i * NS
        pltpu.sync_copy(acc.at[pl.ds(ch * 40, 40), :],
                        out_hbm.at[c, pl.ds(ch * 40, 40), :])
        return carry

    lax.fori_loop(0, nz, fchunk, 0)


RB = 1000


def _tc_body(x_ref, a0_ref, a1_ref, wr_ref, wa0_ref, wa1_ref, b_ref, o_ref):
    acc = jnp.dot(x_ref[...], wr_ref[...], preferred_element_type=jnp.float32)
    acc = acc + jnp.dot(a0_ref[...], wa0_ref[...],
                        preferred_element_type=jnp.float32)
    acc = acc + jnp.dot(a1_ref[...], wa1_ref[...],
                        preferred_element_type=jnp.float32)
    acc = acc + b_ref[...]
    o_ref[...] = jnp.maximum(acc, 0.0).astype(jnp.bfloat16)


_tc_dense = pl.pallas_call(
    _tc_body,
    grid=(N // RB,),
    in_specs=[
        pl.BlockSpec((RB, D), lambda i: (i, 0)),
        pl.BlockSpec((RB, DH), lambda i: (i, 0)),
        pl.BlockSpec((RB, DH), lambda i: (i, 0)),
        pl.BlockSpec((D, D), lambda i: (0, 0)),
        pl.BlockSpec((DH, D), lambda i: (0, 0)),
        pl.BlockSpec((DH, D), lambda i: (0, 0)),
        pl.BlockSpec((1, D), lambda i: (0, 0)),
    ],
    out_specs=pl.BlockSpec((RB, D), lambda i: (i, 0)),
    out_shape=jax.ShapeDtypeStruct((N, D), jnp.bfloat16),
)


@functools.partial(
    pl.kernel,
    out_type=jax.ShapeDtypeStruct((NF, D), jnp.float32),
    mesh=_mesh(),
    compiler_params=pltpu.CompilerParams(use_tc_tiling_on_sc=False,
                                         needs_layout_passes=False),
    scratch_types=[
        pltpu.VMEM((CPT_C, CH), jnp.int32),
        pltpu.VMEM((CPT_C, CH), jnp.int32),
        pltpu.VMEM((CPT_C * CH,), jnp.float32),
        pltpu.VMEM((CH, 32), jnp.bfloat16),
        pltpu.VMEM((CH, 32), jnp.bfloat16),
        pltpu.VMEM((CH, 32), jnp.float32),
        pltpu.VMEM((CH, 32), jnp.float32),
        pltpu.VMEM((200, 32), jnp.float32),
        pltpu.VMEM_SHARED((NF, 32), jnp.float32),
        pltpu.SemaphoreType.DMA,
        pltpu.SemaphoreType.DMA,
        pltpu.SemaphoreType.DMA,
        pltpu.SemaphoreType.DMA,
    ],
)
def _sc_unpool(h_hbm, ps_hbm, fid_hbm, wp_hbm, out_hbm,
               gi_v, fi_v, wp_v, rows0_v, rows1_v, srows0_v, srows1_v,
               zero_v, acc, gsem0, gsem1, ssem0, ssem1):
    c = lax.axis_index("c")
    s = lax.axis_index("s")
    rows = (rows0_v, rows1_v)
    srows = (srows0_v, srows1_v)
    gsem = (gsem0, gsem1)
    ssem = (ssem0, ssem1)

    zf = jnp.zeros((L,), jnp.float32)

    def zrow(r, carry):
        zero_v[r, pl.ds(0, L)] = zf
        zero_v[r, pl.ds(L, L)] = zf
        return carry

    lax.fori_loop(0, 200, zrow, 0)
    nz = (NF // 200 - s + NS - 1) // NS

    def zchunk(i, carry):
        ch = s + i * NS
        pltpu.sync_copy(zero_v, acc.at[pl.ds(ch * 200, 200), :])
        return carry

    pltpu.sync_copy(fid_hbm.at[s], fi_v)
    pltpu.sync_copy(wp_hbm.at[s], wp_v)

    def gather(ci, b):
        return pltpu.async_copy(h_hbm.at[gi_v.at[ci]], rows[b], gsem[b])

    def scatter(ci, b):
        return pltpu.async_copy(srows[b], acc.at[fi_v.at[ci]], ssem[b],
                                add=True)

    for blk in range(2):
        cb = c * 2 + blk
        lax.fori_loop(0, nz, zchunk, 0)
        pltpu.sync_copy(ps_hbm.at[s], gi_v)
        offv = lax.broadcast(cb, (L,))

        def shift(r, carry):
            for k in range(CH // L):
                gi_v[r, pl.ds(k * L, L)] = (
                    gi_v[r, pl.ds(k * L, L)] * 4 + offv)
            return carry

        lax.fori_loop(0, CPT_C, shift, 0)
        plsc.subcore_barrier()

        gather(0, 0)
        gather(1, 1)

        def pair(t, carry):
            for b in range(2):
                ci = 2 * t + b
                pltpu.make_async_copy(h_hbm.at[gi_v.at[ci]], rows[b],
                                      gsem[b]).wait()

                @pl.when(t > 0)
                def _drain():
                    pltpu.make_async_copy(srows[b], acc.at[fi_v.at[ci]],
                                          ssem[b]).wait()

                def subgroup(q, ecarry):
                    wq = wp_v[pl.ds(ci * CH + q * L, L)]
                    for j2 in range(L):
                        bw = lax.broadcast(wq[j2], (L,))
                        j = q * L + j2
                        p = rows[b][j, pl.ds(0, 2 * L)]
                        ae, ao = plsc.unpack(
                            p, format=plsc.PackFormat.INTERLEAVED)
                        srows[b][j, pl.ds(0, L)] = ae * bw
                        srows[b][j, pl.ds(L, L)] = ao * bw
                    return ecarry

                lax.fori_loop(0, CH // L, subgroup, 0)
                scatter(ci, b)

                @pl.when(t < (CPT_C // 2) - 1)
                def _prefetch():
                    gather(ci + 2, b)
            return carry

        lax.fori_loop(0, CPT_C // 2, pair, 0)
        for b in range(2):
            pltpu.make_async_copy(srows[b], acc.at[fi_v.at[CPT_C - 2 + b]],
                                  ssem[b]).wait()
        plsc.subcore_barrier()

        def fchunk(i, carry):
            ch = s + i * NS
            pltpu.sync_copy(acc.at[pl.ds(ch * 200, 200), :],
                            out_hbm.at[pl.ds(ch * 200, 200),
                                       pl.ds(cb * 32, 32)])
            return carry

        lax.fori_loop(0, nz, fchunk, 0)
        plsc.subcore_barrier()


def kernel(x, edge_index, edge_attr, pool_edge_index, pool_edge_attr,
           n_fine, W_root, W_agg, b, we):
    pe = EPAD - E
    src = jnp.pad(edge_index[0], (0, pe)).reshape(NS, CPT_A, CH)
    dst = jnp.pad(edge_index[1], (0, pe)).reshape(NS, CPT_A, CH)
    ea = jnp.pad(edge_attr.T, ((0, 0), (0, pe))).reshape(4, NS, CPT_A * CH)
    we16 = jnp.zeros((16,), jnp.float32).at[:4].set(we[:, 0])
    xcs = x.astype(jnp.bfloat16).reshape(NC * N, DH)

    a01 = _sc_aggr(xcs, src, dst, ea, we16)
    perm64 = jnp.array(
        [blk * 32 + (2 * r if r < 16 else 2 * (r - 16) + 1)
         for blk in range(2) for r in range(32)], dtype=jnp.int32)
    perm128 = jnp.array(
        [32 * g + (jj // 2) + (16 if jj % 2 else 0)
         for g in range(4) for jj in range(32)], dtype=jnp.int32)
    h = _tc_dense(x, a01[0], a01[1], W_root[:, perm128],
                  W_agg[perm64][:, perm128], W_agg[DH + perm64][:, perm128],
                  b[perm128].reshape(1, D))

    hcb = h.reshape(4 * N, 32)
    pp = EPPAD - EP
    ps = jnp.pad(pool_edge_index[1], (0, pp)).reshape(NS, CPT_C, CH)
    fid = jnp.minimum(pool_edge_index[0], n_fine - 1).astype(jnp.int32)
    fid = jnp.pad(fid, (0, pp)).reshape(NS, CPT_C, CH)
    wp = jnp.pad(pool_edge_attr[:, 0], (0, pp)).reshape(NS, CPT_C * CH)
    return _sc_unpool(hcb, ps, fid, wp)
